# baseline (speedup 1.0000x reference)
import functools

import jax
import jax.numpy as jnp
from jax import lax
from jax.experimental import pallas as pl
from jax.experimental.pallas import tpu as pltpu
from jax.experimental.pallas import tpu_sc as plsc

N = 10000
E = 320000
NC = 2
NS = 16
CHUNK = 64
NB = 4


def _make_seg_sum(d, col_split):
  edges_per_core = E if col_split else E // 2
  ept = edges_per_core // NS
  nchunk, tail = divmod(ept, CHUNK)
  nblk, remb = divmod(nchunk, NB)
  assert remb == 0 and tail % 8 == 0
  rpt = 632
  n_acc = (NS - 1) * rpt + 528
  last_wb = N - (NS - 1) * rpt
  mesh = plsc.VectorSubcoreMesh(core_axis_name="c", subcore_axis_name="s",
                                num_cores=NC, num_subcores=NS)

  @functools.partial(
      pl.kernel,
      out_type=jax.ShapeDtypeStruct((2 * N, d), jnp.float32),
      mesh=mesh,
      scratch_types=[
          [[pltpu.VMEM((CHUNK,), jnp.int32)] * NB for _ in range(2)],
          [[pltpu.VMEM((CHUNK,), jnp.int32)] * NB for _ in range(2)],
          [pltpu.VMEM((CHUNK, d), jnp.float32)] * NB,
          pltpu.VMEM((tail,), jnp.int32),
          pltpu.VMEM((tail,), jnp.int32),
          pltpu.VMEM_SHARED(((NS - 1) * 632 + 528, d), jnp.float32),
          [[pltpu.SemaphoreType.DMA] * NB for _ in range(2)],
          [pltpu.SemaphoreType.DMA] * NB,
          [pltpu.SemaphoreType.DMA] * NB,
      ],
  )
  def seg(src_hbm, dst_hbm, zeros_hbm, table_hbm, out_hbm,
          src_v, dst_v, rows_v, src_t, dst_t, agg, isem, gsem, ssem):
    c = lax.axis_index("c")
    s = lax.axis_index("s")
    if col_split:
      ebase = s * ept
      soff = c * E
    else:
      ebase = c * (E // 2) + s * ept
      soff = 0

    def fire_idx(g, blk_e0):
      for b in range(NB):
        eb = lax.min(blk_e0 + b * CHUNK, E - CHUNK)
        pltpu.async_copy(src_hbm.at[pl.ds(soff + eb, CHUNK)],
                         src_v[g][b], isem[g][b])
        pltpu.async_copy(dst_hbm.at[pl.ds(eb, CHUNK)], dst_v[g][b], isem[g][b])

    def wait_idx(g, b):
      pltpu.make_async_copy(src_hbm.at[pl.ds(0, CHUNK)],
                            src_v[g][b], isem[g][b]).wait()
      pltpu.make_async_copy(dst_hbm.at[pl.ds(0, CHUNK)],
                            dst_v[g][b], isem[g][b]).wait()

    def wait_scatter(g, b):
      pltpu.make_async_copy(rows_v[b], agg.at[dst_v[g][b]], ssem[b]).wait()

    def sub_block(j, g, blk, skip_first_reclaim=False):
      e0 = ebase + blk * (NB * CHUNK)
      gather_cps = []
      for b in range(NB):
        if skip_first_reclaim:
          @pl.when(j > 0)
          def _reclaim(b=b):
            wait_scatter(1 - g, b)
        else:
          wait_scatter(1 - g, b)
        wait_idx(g, b)
        gather_cps.append(
            pltpu.async_copy(table_hbm.at[src_v[g][b]], rows_v[b], gsem[b]))
      fire_idx(1 - g, e0 + NB * CHUNK)
      for b in range(NB):
        gather_cps[b].wait()
        pltpu.async_copy(rows_v[b], agg.at[dst_v[g][b]], ssem[b], add=True)

    def body(j, carry):
      sub_block(j, 0, 2 * j, skip_first_reclaim=True)
      sub_block(j, 1, 2 * j + 1)
      return carry

    fire_idx(0, ebase)

    @pl.when(s < NS - 1)
    def _zero_full():
      pltpu.sync_copy(zeros_hbm.at[pl.ds(s * rpt, rpt)],
                      agg.at[pl.ds(s * rpt, rpt)])

    @pl.when(s == NS - 1)
    def _zero_last():
      pltpu.sync_copy(zeros_hbm.at[pl.ds((NS - 1) * rpt, 528)],
                      agg.at[pl.ds((NS - 1) * rpt, 528)])

    plsc.subcore_barrier()

    lax.fori_loop(0, nblk // 2, body, 0)
    if nblk % 2:
      sub_block(1, 0, nblk - 1)
      last_g = 0
    else:
      last_g = 1
    for b in range(NB):
      wait_scatter(last_g, b)
    for b in range(NB):
      wait_idx(1 - last_g, b)

    if tail:
      et = ebase + nchunk * CHUNK
      cs = pltpu.async_copy(src_hbm.at[pl.ds(soff + et, tail)], src_t,
                            isem[0][0])
      cd = pltpu.async_copy(dst_hbm.at[pl.ds(et, tail)], dst_t, isem[0][0])
      cs.wait()
      cd.wait()
      pltpu.async_copy(table_hbm.at[src_t], rows_v[0].at[pl.ds(0, tail)],
                       gsem[0]).wait()
      pltpu.async_copy(rows_v[0].at[pl.ds(0, tail)], agg.at[dst_t], ssem[0],
                       add=True).wait()

    plsc.subcore_barrier()

    @pl.when(s < NS - 1)
    def _full_tile_writeback():
      pltpu.sync_copy(agg.at[pl.ds(s * rpt, rpt)],
                      out_hbm.at[pl.ds(c * N + s * rpt, rpt)])

    @pl.when(s == NS - 1)
    def _last_tile_writeback():
      pltpu.sync_copy(agg.at[pl.ds((NS - 1) * rpt, last_wb)],
                      out_hbm.at[pl.ds(c * N + (NS - 1) * rpt, last_wb)])

  return seg


_seg1 = _make_seg_sum(128, col_split=False)
_seg2 = _make_seg_sum(128, col_split=True)

_R = 1000


def _root1_body(x_ref, wt_ref, b_ref, o_ref):
  o_ref[...] = (
      jnp.dot(x_ref[...], wt_ref[...], preferred_element_type=jnp.float32)
      + b_ref[...])


def _root1(x, w_root_t, b_rel):
  nb = N // _R
  return pl.pallas_call(
      _root1_body,
      grid=(nb,),
      in_specs=[
          pl.BlockSpec((_R, 128), lambda i: (i, 0)),
          pl.BlockSpec((128, 256), lambda i: (0, 0)),
          pl.BlockSpec((1, 256), lambda i: (0, 0)),
      ],
      out_specs=pl.BlockSpec((_R, 256), lambda i: (i, 0)),
      out_shape=jax.ShapeDtypeStruct((N, 256), jnp.float32),
  )(x, w_root_t, b_rel)


def _root2_body(h0_ref, h1_ref, wt_ref, b_ref, o_ref):
  z = jnp.dot(h0_ref[...], wt_ref[:128], preferred_element_type=jnp.float32)
  z = z + jnp.dot(h1_ref[...], wt_ref[128:], preferred_element_type=jnp.float32)
  o_ref[...] = z + b_ref[...]


def _root2(h_cat, w_root_t, b_rel):
  nb = N // _R
  return pl.pallas_call(
      _root2_body,
      grid=(nb,),
      in_specs=[
          pl.BlockSpec((_R, 128), lambda i: (i, 0)),
          pl.BlockSpec((_R, 128), lambda i: (i + nb, 0)),
          pl.BlockSpec((256, 256), lambda i: (0, 0)),
          pl.BlockSpec((1, 256), lambda i: (0, 0)),
      ],
      out_specs=pl.BlockSpec((_R, 256), lambda i: (i, 0)),
      out_shape=jax.ShapeDtypeStruct((N, 256), jnp.float32),
  )(h_cat, h_cat, w_root_t, b_rel)


def _tc1_body(r_ref, p0_ref, p1_ref, wr_ref, o_ref):
  agg = p0_ref[...] + p1_ref[...]
  z = jnp.dot(agg, wr_ref[...], preferred_element_type=jnp.float32)
  z = jnp.maximum(z + r_ref[...], 0.0)
  o_ref[0] = z[:, :128]
  o_ref[1] = z[:, 128:]


def _tc1(root1, p_cat, w_rel_t):
  nb = N // _R
  return pl.pallas_call(
      _tc1_body,
      grid=(nb,),
      in_specs=[
          pl.BlockSpec((_R, 256), lambda i: (i, 0)),
          pl.BlockSpec((_R, 128), lambda i: (i, 0)),
          pl.BlockSpec((_R, 128), lambda i: (i + nb, 0)),
          pl.BlockSpec((128, 256), lambda i: (0, 0)),
      ],
      out_specs=pl.BlockSpec((2, _R, 128), lambda i: (0, i, 0)),
      out_shape=jax.ShapeDtypeStruct((2, N, 128), jnp.float32),
  )(root1, p_cat, p_cat, w_rel_t)


def _tc2_body(r_ref, a0_ref, a1_ref, wr_ref, o_ref):
  z = jnp.dot(a0_ref[...], wr_ref[:128], preferred_element_type=jnp.float32)
  z = z + jnp.dot(a1_ref[...], wr_ref[128:], preferred_element_type=jnp.float32)
  o_ref[...] = jnp.maximum(z + r_ref[...], 0.0)


def _tc2(root2, a_cat, w_rel_t):
  nb = N // _R
  return pl.pallas_call(
      _tc2_body,
      grid=(nb,),
      in_specs=[
          pl.BlockSpec((_R, 256), lambda i: (i, 0)),
          pl.BlockSpec((_R, 128), lambda i: (i, 0)),
          pl.BlockSpec((_R, 128), lambda i: (i + nb, 0)),
          pl.BlockSpec((256, 256), lambda i: (0, 0)),
      ],
      out_specs=pl.BlockSpec((_R, 256), lambda i: (i, 0)),
      out_shape=jax.ShapeDtypeStruct((N, 256), jnp.float32),
  )(root2, a_cat, a_cat, w_rel_t)


def kernel(x, edge_index, W1_rel, b1_rel, W1_root, W2_rel, b2_rel, W2_root):
  ei = edge_index.astype(jnp.int32)
  src, dst = ei[0], ei[1]
  src1 = src
  src2 = jnp.concatenate([src, src + N])
  zeros = jnp.zeros(((NS - 1) * 632 + 528, 128), jnp.float32)

  p_cat = _seg1(src1, dst, zeros, x)
  root1 = _root1(x, W1_root.T, b1_rel[None])
  h_cat = _tc1(root1, p_cat, W1_rel.T).reshape(2 * N, 128)
  a_cat = _seg2(src2, dst, zeros, h_cat)
  root2 = _root2(h_cat, W2_root.T, b2_rel[None])
  return _tc2(root2, a_cat, W2_rel.T)


# speedup vs baseline: 1.0021x; 1.0021x over previous
import functools

import jax
import jax.numpy as jnp
from jax import lax
from jax.experimental import pallas as pl
from jax.experimental.pallas import tpu as pltpu
from jax.experimental.pallas import tpu_sc as plsc

N = 10000
E = 320000
NC = 2
NS = 16
CHUNK = 64
NB = 4


def _make_seg_sum(d, col_split):
  edges_per_core = E if col_split else E // 2
  ept = edges_per_core // NS
  nchunk, tail = divmod(ept, CHUNK)
  nblk, remb = divmod(nchunk, NB)
  assert remb == 0 and tail % 8 == 0
  rpt = 632
  n_acc = (NS - 1) * rpt + 528
  last_wb = N - (NS - 1) * rpt
  mesh = plsc.VectorSubcoreMesh(core_axis_name="c", subcore_axis_name="s",
                                num_cores=NC, num_subcores=NS)

  @functools.partial(
      pl.kernel,
      out_type=jax.ShapeDtypeStruct((2 * N, d), jnp.float32),
      mesh=mesh,
      scratch_types=[
          [[pltpu.VMEM((CHUNK,), jnp.int32)] * NB for _ in range(2)],
          [[pltpu.VMEM((CHUNK,), jnp.int32)] * NB for _ in range(2)],
          [pltpu.VMEM((CHUNK, d), jnp.float32)] * NB,
          pltpu.VMEM((tail,), jnp.int32),
          pltpu.VMEM((tail,), jnp.int32),
          pltpu.VMEM_SHARED(((NS - 1) * 632 + 528, d), jnp.float32),
          [[pltpu.SemaphoreType.DMA] * NB for _ in range(2)],
          [pltpu.SemaphoreType.DMA] * NB,
          [pltpu.SemaphoreType.DMA] * NB,
      ],
  )
  def seg(src_hbm, dst_hbm, zeros_hbm, table_hbm, out_hbm,
          src_v, dst_v, rows_v, src_t, dst_t, agg, isem, gsem, ssem):
    c = lax.axis_index("c")
    s = lax.axis_index("s")
    if col_split:
      ebase = s * ept
      soff = c * E
    else:
      ebase = c * (E // 2) + s * ept
      soff = 0

    def fire_idx(g, blk_e0):
      for b in range(NB):
        eb = lax.min(blk_e0 + b * CHUNK, E - CHUNK)
        pltpu.async_copy(src_hbm.at[pl.ds(soff + eb, CHUNK)],
                         src_v[g][b], isem[g][b])
        pltpu.async_copy(dst_hbm.at[pl.ds(eb, CHUNK)], dst_v[g][b], isem[g][b])

    def wait_idx(g, b):
      pltpu.make_async_copy(src_hbm.at[pl.ds(0, CHUNK)],
                            src_v[g][b], isem[g][b]).wait()
      pltpu.make_async_copy(dst_hbm.at[pl.ds(0, CHUNK)],
                            dst_v[g][b], isem[g][b]).wait()

    def wait_scatter(g, b):
      pltpu.make_async_copy(rows_v[b], agg.at[dst_v[g][b]], ssem[b]).wait()

    def sub_block(j, g, blk, skip_first_reclaim=False):
      e0 = ebase + blk * (NB * CHUNK)
      gather_cps = []
      for b in range(NB):
        if skip_first_reclaim:
          @pl.when(j > 0)
          def _reclaim(b=b):
            wait_scatter(1 - g, b)
        else:
          wait_scatter(1 - g, b)
        wait_idx(g, b)
        gather_cps.append(
            pltpu.async_copy(table_hbm.at[src_v[g][b]], rows_v[b], gsem[b]))
      fire_idx(1 - g, e0 + NB * CHUNK)
      for b in range(NB):
        gather_cps[b].wait()
        pltpu.async_copy(rows_v[b], agg.at[dst_v[g][b]], ssem[b], add=True)

    def body(j, carry):
      sub_block(j, 0, 2 * j, skip_first_reclaim=True)
      sub_block(j, 1, 2 * j + 1)
      return carry

    fire_idx(0, ebase)

    @pl.when(s < NS - 1)
    def _zero_full():
      pltpu.sync_copy(zeros_hbm.at[pl.ds(s * rpt, rpt)],
                      agg.at[pl.ds(s * rpt, rpt)])

    @pl.when(s == NS - 1)
    def _zero_last():
      pltpu.sync_copy(zeros_hbm.at[pl.ds((NS - 1) * rpt, 528)],
                      agg.at[pl.ds((NS - 1) * rpt, 528)])

    plsc.subcore_barrier()

    lax.fori_loop(0, nblk // 2, body, 0)
    if nblk % 2:
      sub_block(1, 0, nblk - 1)
      last_g = 0
    else:
      last_g = 1
    for b in range(NB):
      wait_scatter(last_g, b)
    for b in range(NB):
      wait_idx(1 - last_g, b)

    if tail:
      et = ebase + nchunk * CHUNK
      cs = pltpu.async_copy(src_hbm.at[pl.ds(soff + et, tail)], src_t,
                            isem[0][0])
      cd = pltpu.async_copy(dst_hbm.at[pl.ds(et, tail)], dst_t, isem[0][0])
      cs.wait()
      cd.wait()
      pltpu.async_copy(table_hbm.at[src_t], rows_v[0].at[pl.ds(0, tail)],
                       gsem[0]).wait()
      pltpu.async_copy(rows_v[0].at[pl.ds(0, tail)], agg.at[dst_t], ssem[0],
                       add=True).wait()

    plsc.subcore_barrier()

    @pl.when(s < NS - 1)
    def _full_tile_writeback():
      pltpu.sync_copy(agg.at[pl.ds(s * rpt, rpt)],
                      out_hbm.at[pl.ds(c * N + s * rpt, rpt)])

    @pl.when(s == NS - 1)
    def _last_tile_writeback():
      pltpu.sync_copy(agg.at[pl.ds((NS - 1) * rpt, last_wb)],
                      out_hbm.at[pl.ds(c * N + (NS - 1) * rpt, last_wb)])

  return seg


_seg1 = _make_seg_sum(128, col_split=False)
_seg2 = _make_seg_sum(128, col_split=True)

_R = 1000


def _tc1_body(x_ref, p0_ref, p1_ref, wr_ref, b_ref, wt_ref, o_ref):
  agg = p0_ref[...] + p1_ref[...]
  z = jnp.dot(agg, wr_ref[...], preferred_element_type=jnp.float32)
  z = z + jnp.dot(x_ref[...], wt_ref[...], preferred_element_type=jnp.float32)
  z = jnp.maximum(z + b_ref[...], 0.0)
  o_ref[0] = z[:, :128]
  o_ref[1] = z[:, 128:]


def _tc1(x, p_cat, w_rel_t, b_rel, w_root_t):
  nb = N // _R
  return pl.pallas_call(
      _tc1_body,
      grid=(nb,),
      in_specs=[
          pl.BlockSpec((_R, 128), lambda i: (i, 0)),
          pl.BlockSpec((_R, 128), lambda i: (i, 0)),
          pl.BlockSpec((_R, 128), lambda i: (i + nb, 0)),
          pl.BlockSpec((128, 256), lambda i: (0, 0)),
          pl.BlockSpec((1, 256), lambda i: (0, 0)),
          pl.BlockSpec((128, 256), lambda i: (0, 0)),
      ],
      out_specs=pl.BlockSpec((2, _R, 128), lambda i: (0, i, 0)),
      out_shape=jax.ShapeDtypeStruct((2, N, 128), jnp.float32),
  )(x, p_cat, p_cat, w_rel_t, b_rel, w_root_t)


def _tc2_body(h0_ref, h1_ref, a0_ref, a1_ref, wr_ref, b_ref, wt_ref, o_ref):
  z = jnp.dot(a0_ref[...], wr_ref[:128], preferred_element_type=jnp.float32)
  z = z + jnp.dot(a1_ref[...], wr_ref[128:], preferred_element_type=jnp.float32)
  z = z + jnp.dot(h0_ref[...], wt_ref[:128], preferred_element_type=jnp.float32)
  z = z + jnp.dot(h1_ref[...], wt_ref[128:], preferred_element_type=jnp.float32)
  o_ref[...] = jnp.maximum(z + b_ref[...], 0.0)


def _tc2(h_cat, a_cat, w_rel_t, b_rel, w_root_t):
  nb = N // _R
  return pl.pallas_call(
      _tc2_body,
      grid=(nb,),
      in_specs=[
          pl.BlockSpec((_R, 128), lambda i: (i, 0)),
          pl.BlockSpec((_R, 128), lambda i: (i + nb, 0)),
          pl.BlockSpec((_R, 128), lambda i: (i, 0)),
          pl.BlockSpec((_R, 128), lambda i: (i + nb, 0)),
          pl.BlockSpec((256, 256), lambda i: (0, 0)),
          pl.BlockSpec((1, 256), lambda i: (0, 0)),
          pl.BlockSpec((256, 256), lambda i: (0, 0)),
      ],
      out_specs=pl.BlockSpec((_R, 256), lambda i: (i, 0)),
      out_shape=jax.ShapeDtypeStruct((N, 256), jnp.float32),
  )(h_cat, h_cat, a_cat, a_cat, w_rel_t, b_rel, w_root_t)


def kernel(x, edge_index, W1_rel, b1_rel, W1_root, W2_rel, b2_rel, W2_root):
  ei = edge_index.astype(jnp.int32)
  src, dst = ei[0], ei[1]
  src1 = src
  src2 = jnp.concatenate([src, src + N])
  zeros = jnp.zeros(((NS - 1) * 632 + 528, 128), jnp.float32)

  p_cat = _seg1(src1, dst, zeros, x)
  h_cat = _tc1(x, p_cat, W1_rel.T, b1_rel[None], W1_root.T).reshape(2 * N, 128)
  a_cat = _seg2(src2, dst, zeros, h_cat)
  return _tc2(h_cat, a_cat, W2_rel.T, b2_rel[None], W2_root.T)
